# baseline (speedup 1.0000x reference)
import jax
import jax.numpy as jnp
from jax import lax
from jax.experimental import pallas as pl
from jax.experimental.pallas import tpu as pltpu

BATCH = 4
SEQ = 4096
D_MODEL = 2048
ROWS = BATCH * SEQ

SIZES = (64, 192, 256, 512, 512, 512, 512, 512, 512, 256, 192, 64)
OFFS = tuple(sum(SIZES[:j]) for j in range(len(SIZES)))
PER_B = len(SIZES)
MAXC = max(SIZES)
NBUF = 4
assert sum(SIZES) == SEQ and PER_B % NBUF == 0


def _body(ids_ref, x_hbm, tab_hbm, o_hbm, rows_v, in_buf, out_buf,
          row_sem, in_sems, out_sems):
    def in_copy(b, j, k):
        return pltpu.make_async_copy(
            x_hbm.at[pl.ds(b * SEQ + OFFS[j], SIZES[j]), :],
            in_buf.at[k, pl.ds(0, SIZES[j]), :], in_sems.at[k])

    def out_copy(b, j, k):
        return pltpu.make_async_copy(
            out_buf.at[k, pl.ds(0, SIZES[j]), :],
            o_hbm.at[pl.ds(b * SEQ + OFFS[j], SIZES[j]), :], out_sems.at[k])

    for b in range(BATCH):
        pltpu.make_async_copy(
            tab_hbm.at[pl.ds(ids_ref[b], 1), :], rows_v.at[pl.ds(b, 1), :],
            row_sem).start()
    for j in range(NBUF):
        in_copy(0, j, j).start()
    for b in range(BATCH):
        pltpu.make_async_copy(
            tab_hbm.at[pl.ds(ids_ref[b], 1), :], rows_v.at[pl.ds(b, 1), :],
            row_sem).wait()

    def batch_step(b, _):
        for j in range(PER_B):
            k = j % NBUF
            in_copy(b, j, k).wait()
            if j >= NBUF:
                out_copy(b, j - NBUF, k).wait()
            else:
                @pl.when(b > 0)
                def _(b=b, j=j, k=k):
                    out_copy(b - 1, j - NBUF + PER_B, k).wait()
            out_buf[k, pl.ds(0, SIZES[j]), :] = (
                in_buf[k, pl.ds(0, SIZES[j]), :] + rows_v[pl.ds(b, 1), :])
            out_copy(b, j, k).start()
            if j + NBUF < PER_B:
                in_copy(b, j + NBUF, k).start()
            else:
                @pl.when(b < BATCH - 1)
                def _(b=b, j=j, k=k):
                    in_copy(b + 1, j + NBUF - PER_B, k).start()
        return 0

    lax.fori_loop(0, BATCH, batch_step, 0)
    for j in range(PER_B - NBUF, PER_B):
        out_copy(BATCH - 1, j, j % NBUF).wait()


def kernel(x, code_ids, control_table):
    grid_spec = pltpu.PrefetchScalarGridSpec(
        num_scalar_prefetch=1,
        grid=(1,),
        in_specs=[
            pl.BlockSpec(memory_space=pl.ANY),
            pl.BlockSpec(memory_space=pl.ANY),
        ],
        out_specs=pl.BlockSpec(memory_space=pl.ANY),
        scratch_shapes=[
            pltpu.VMEM((BATCH, D_MODEL), jnp.float32),
            pltpu.VMEM((NBUF, MAXC, D_MODEL), jnp.float32),
            pltpu.VMEM((NBUF, MAXC, D_MODEL), jnp.float32),
            pltpu.SemaphoreType.DMA,
            pltpu.SemaphoreType.DMA((NBUF,)),
            pltpu.SemaphoreType.DMA((NBUF,)),
        ],
    )
    out = pl.pallas_call(
        _body,
        grid_spec=grid_spec,
        out_shape=jax.ShapeDtypeStruct((ROWS, D_MODEL), x.dtype),
    )(code_ids.astype(jnp.int32), x.reshape(ROWS, D_MODEL), control_table)
    return out.reshape(x.shape)


# speedup vs baseline: 1.0089x; 1.0089x over previous
import jax
import jax.numpy as jnp
from jax import lax
from jax.experimental import pallas as pl
from jax.experimental.pallas import tpu as pltpu

BATCH = 4
SEQ = 4096
D_MODEL = 2048
ROWS = BATCH * SEQ

SIZES = (256, 1024, 1024, 1024, 512, 256)
OFFS = tuple(sum(SIZES[:j]) for j in range(len(SIZES)))
PER_B = len(SIZES)
MAXC = max(SIZES)
NBUF = 3
assert sum(SIZES) == SEQ and PER_B % NBUF == 0


def _body(ids_ref, x_hbm, tab_hbm, o_hbm, rows_v, in_buf, out_buf,
          row_sem, in_sems, out_sems):
    def in_copy(b, j, k):
        return pltpu.make_async_copy(
            x_hbm.at[pl.ds(b * SEQ + OFFS[j], SIZES[j]), :],
            in_buf.at[k, pl.ds(0, SIZES[j]), :], in_sems.at[k])

    def out_copy(b, j, k):
        return pltpu.make_async_copy(
            out_buf.at[k, pl.ds(0, SIZES[j]), :],
            o_hbm.at[pl.ds(b * SEQ + OFFS[j], SIZES[j]), :], out_sems.at[k])

    for b in range(BATCH):
        pltpu.make_async_copy(
            tab_hbm.at[pl.ds(ids_ref[b], 1), :], rows_v.at[pl.ds(b, 1), :],
            row_sem).start()
    for j in range(NBUF):
        in_copy(0, j, j).start()
    for b in range(BATCH):
        pltpu.make_async_copy(
            tab_hbm.at[pl.ds(ids_ref[b], 1), :], rows_v.at[pl.ds(b, 1), :],
            row_sem).wait()

    def batch_step(b, _):
        for j in range(PER_B):
            k = j % NBUF
            in_copy(b, j, k).wait()
            if j >= NBUF:
                out_copy(b, j - NBUF, k).wait()
            else:
                @pl.when(b > 0)
                def _(b=b, j=j, k=k):
                    out_copy(b - 1, j - NBUF + PER_B, k).wait()
            out_buf[k, pl.ds(0, SIZES[j]), :] = (
                in_buf[k, pl.ds(0, SIZES[j]), :] + rows_v[pl.ds(b, 1), :])
            out_copy(b, j, k).start()
            if j + NBUF < PER_B:
                in_copy(b, j + NBUF, k).start()
            else:
                @pl.when(b < BATCH - 1)
                def _(b=b, j=j, k=k):
                    in_copy(b + 1, j + NBUF - PER_B, k).start()
        return 0

    lax.fori_loop(0, BATCH, batch_step, 0)
    for j in range(PER_B - NBUF, PER_B):
        out_copy(BATCH - 1, j, j % NBUF).wait()


def kernel(x, code_ids, control_table):
    grid_spec = pltpu.PrefetchScalarGridSpec(
        num_scalar_prefetch=1,
        grid=(1,),
        in_specs=[
            pl.BlockSpec(memory_space=pl.ANY),
            pl.BlockSpec(memory_space=pl.ANY),
        ],
        out_specs=pl.BlockSpec(memory_space=pl.ANY),
        scratch_shapes=[
            pltpu.VMEM((BATCH, D_MODEL), jnp.float32),
            pltpu.VMEM((NBUF, MAXC, D_MODEL), jnp.float32),
            pltpu.VMEM((NBUF, MAXC, D_MODEL), jnp.float32),
            pltpu.SemaphoreType.DMA,
            pltpu.SemaphoreType.DMA((NBUF,)),
            pltpu.SemaphoreType.DMA((NBUF,)),
        ],
    )
    out = pl.pallas_call(
        _body,
        grid_spec=grid_spec,
        out_shape=jax.ShapeDtypeStruct((ROWS, D_MODEL), x.dtype),
    )(code_ids.astype(jnp.int32), x.reshape(ROWS, D_MODEL), control_table)
    return out.reshape(x.shape)
